# Initial kernel scaffold; baseline (speedup 1.0000x reference)
import jax
import jax.numpy as jnp
from jax.experimental import pallas as pl


def kernel(target_samples, X, neigh_idx, W1, W2, W, b):
    raise NotImplementedError("write your pallas kernel here")


# speedup vs baseline: 1.1453x; 1.1453x over previous
import functools

import jax
import jax.numpy as jnp
from jax import lax
from jax.experimental import pallas as pl
from jax.experimental.pallas import tpu as pltpu
from jax.experimental.pallas import tpu_sc as plsc

N_NODES = 10000
M = 32
D = 128
NN = 100
N_GRAPHS = 64


def _make_sc_segment_sum(n_out, S, Sp, C, n_active=None):
    info = plsc.get_sparse_core_info()
    NC, NS = info.num_cores, info.num_subcores
    NW = NC * NS
    if n_active is None:
        n_active = NW
    assert n_out % n_active == 0
    R = n_out // n_active
    assert R % C == 0 and R % 8 == 0
    n_chunks = R // C
    G = (C * Sp) // 128
    assert C * Sp % 128 == 0
    n_grp = (n_out * Sp) // 128
    grp_per_worker = (R * Sp) // 128
    whole_idx = (grp_per_worker % 8) != 0
    idx_rows = n_grp if whole_idx else grp_per_worker

    mesh = plsc.VectorSubcoreMesh(core_axis_name="c", subcore_axis_name="s")

    @functools.partial(
        pl.kernel,
        mesh=mesh,
        out_type=jax.ShapeDtypeStruct((n_out, D), jnp.float32),
        scratch_types=[
            pltpu.VMEM((idx_rows, 128), jnp.int32),
            pltpu.VMEM((C * Sp, D), jnp.float32),
            pltpu.VMEM((R, D), jnp.float32),
            pltpu.SemaphoreType.DMA,
        ],
    )
    def seg_sum(table_hbm, idx_hbm, out_hbm, idx_v, rows_v, out_v, sem):
        cid = lax.axis_index("c")
        sid = lax.axis_index("s")
        wid = sid * NC + cid

        def work():
            if whole_idx:
                pltpu.sync_copy(idx_hbm, idx_v)
                grp_off = wid * grp_per_worker
            else:
                pltpu.sync_copy(
                    idx_hbm.at[pl.ds(wid * grp_per_worker, grp_per_worker)],
                    idx_v,
                )
                grp_off = 0

            def chunk(t, carry):
                copies = [
                    pltpu.async_copy(
                        table_hbm.at[idx_v.at[grp_off + t * G + g]],
                        rows_v.at[pl.ds(g * 128, 128)],
                        sem,
                    )
                    for g in range(G)
                ]
                for cp in copies:
                    cp.wait()

                def node(n, carry2):
                    rbase = n * Sp
                    for k in range(D // 16):
                        sl = pl.ds(k * 16, 16)
                        acc = rows_v[rbase, sl]
                        for j in range(1, S):
                            acc = acc + rows_v[rbase + j, sl]
                        out_v[t * C + n, sl] = acc
                    return carry2

                lax.fori_loop(0, C, node, 0)
                return carry

            lax.fori_loop(0, n_chunks, chunk, 0)
            pltpu.sync_copy(out_v, out_hbm.at[pl.ds(wid * R, R)])

        if n_active < NW:
            pl.when(wid < n_active)(work)
        else:
            work()

    return seg_sum


def _mm_relu_body(h_ref, s_ref, wa_ref, wb_ref, o_ref):
    o_ref[...] = jnp.maximum(
        jnp.dot(h_ref[...], wa_ref[...], preferred_element_type=jnp.float32)
        + jnp.dot(s_ref[...], wb_ref[...], preferred_element_type=jnp.float32),
        0.0,
    )


def _mm_relu(h, s, wa, wb, block_rows=512):
    n = h.shape[0]
    grid = n // block_rows
    return pl.pallas_call(
        _mm_relu_body,
        grid=(grid,),
        in_specs=[
            pl.BlockSpec((block_rows, D), lambda i: (i, 0)),
            pl.BlockSpec((block_rows, D), lambda i: (i, 0)),
            pl.BlockSpec((D, D), lambda i: (0, 0)),
            pl.BlockSpec((D, D), lambda i: (0, 0)),
        ],
        out_specs=pl.BlockSpec((block_rows, D), lambda i: (i, 0)),
        out_shape=jax.ShapeDtypeStruct((n, D), jnp.float32),
    )(h, s, wa, wb)


def _cls_body(p_ref, w_ref, b_ref, o_ref):
    logits = (
        jnp.dot(p_ref[...], w_ref[...], preferred_element_type=jnp.float32)
        + b_ref[0:1, :]
    )
    m = jnp.max(logits, axis=1, keepdims=True)
    lse = m + jnp.log(jnp.sum(jnp.exp(logits - m), axis=1, keepdims=True))
    o_ref[...] = logits - lse


def _classifier(p, wpad, bpad):
    return pl.pallas_call(
        _cls_body,
        grid=(1,),
        in_specs=[
            pl.BlockSpec((N_GRAPHS, D), lambda i: (0, 0)),
            pl.BlockSpec((D, D), lambda i: (0, 0)),
            pl.BlockSpec((8, D), lambda i: (0, 0)),
        ],
        out_specs=pl.BlockSpec((N_GRAPHS, D), lambda i: (0, 0)),
        out_shape=jax.ShapeDtypeStruct((N_GRAPHS, D), jnp.float32),
    )(p, wpad, bpad)


def kernel(target_samples, X, neigh_idx, W1, W2, W, b):
    NP = 10240

    Xp = jnp.pad(X, ((0, NP - N_NODES), (0, 0)))
    idx_flat = jnp.pad(
        neigh_idx.astype(jnp.int32).reshape(-1), (0, (NP - N_NODES) * M)
    ).reshape(-1, 128)

    layer_sum = _make_sc_segment_sum(n_out=NP, S=M, Sp=M, C=8)
    pool_sum = _make_sc_segment_sum(n_out=N_GRAPHS, S=NN, Sp=128, C=4, n_active=8)

    s1 = layer_sum(Xp, idx_flat)
    h1 = _mm_relu(Xp, s1, W1[:D], W1[D:] * (1.0 / M))
    s2 = layer_sum(h1, idx_flat)
    h2 = _mm_relu(h1, s2, W2[:D], W2[D:] * (1.0 / M))

    pool_idx = jnp.pad(target_samples.astype(jnp.int32), ((0, 0), (0, 128 - NN)))
    p = pool_sum(h2, pool_idx)

    n_cls = W.shape[1]
    wpad = jnp.zeros((D, D), jnp.float32).at[:, :n_cls].set(W * (1.0 / NN))
    bpad = jnp.full((8, D), -1e30, jnp.float32).at[:, :n_cls].set(b)
    out = _classifier(p, wpad, bpad)
    return out[:, :n_cls]


# speedup vs baseline: 1.3076x; 1.1417x over previous
import functools

import jax
import jax.numpy as jnp
from jax import lax
from jax.experimental import pallas as pl
from jax.experimental.pallas import tpu as pltpu
from jax.experimental.pallas import tpu_sc as plsc

N_NODES = 10000
M = 32
D = 128
NN = 100
N_GRAPHS = 64


def _make_sc_segment_sum(n_out, S, Sp, C, n_active=None):
    info = plsc.get_sparse_core_info()
    NC, NS = info.num_cores, info.num_subcores
    NW = NC * NS
    if n_active is None:
        n_active = NW
    assert n_out % n_active == 0
    R = n_out // n_active
    assert R % C == 0 and R % 8 == 0
    n_chunks = R // C
    G = (C * Sp) // 128
    assert C * Sp % 128 == 0
    n_grp = (n_out * Sp) // 128
    grp_per_worker = (R * Sp) // 128
    whole_idx = (grp_per_worker % 8) != 0
    idx_rows = n_grp if whole_idx else grp_per_worker

    assert n_chunks % 2 == 0
    chunk_writes = (C % 8 == 0)
    out_rows = C if chunk_writes else R
    mesh = plsc.VectorSubcoreMesh(core_axis_name="c", subcore_axis_name="s")

    @functools.partial(
        pl.kernel,
        mesh=mesh,
        out_type=jax.ShapeDtypeStruct((n_out, D), jnp.float32),
        scratch_types=[
            pltpu.VMEM((idx_rows, 128), jnp.int32),
            pltpu.VMEM((C * Sp, D), jnp.float32),
            pltpu.VMEM((C * Sp, D), jnp.float32),
            pltpu.VMEM((out_rows, D), jnp.float32),
            pltpu.SemaphoreType.DMA,
            pltpu.SemaphoreType.DMA,
        ],
    )
    def seg_sum(table_hbm, idx_hbm, out_hbm, idx_v, rows_a, rows_b, out_v,
                sem_a, sem_b):
        cid = lax.axis_index("c")
        sid = lax.axis_index("s")
        wid = sid * NC + cid
        bufs = ((rows_a, sem_a), (rows_b, sem_b))

        def work():
            if whole_idx:
                pltpu.sync_copy(idx_hbm, idx_v)
                grp_off = wid * grp_per_worker
            else:
                pltpu.sync_copy(
                    idx_hbm.at[pl.ds(wid * grp_per_worker, grp_per_worker)],
                    idx_v,
                )
                grp_off = 0

            def fire(t, buf):
                rows, sem = bufs[buf]
                for g in range(G):
                    pltpu.async_copy(
                        table_hbm.at[idx_v.at[grp_off + t * G + g]],
                        rows.at[pl.ds(g * 128, 128)],
                        sem,
                    )

            def drain(buf):
                rows, sem = bufs[buf]
                for g in range(G):
                    pltpu.make_async_copy(
                        table_hbm.at[idx_v.at[grp_off + g]],
                        rows.at[pl.ds(g * 128, 128)],
                        sem,
                    ).wait()

            def compute(t, buf):
                rows, _ = bufs[buf]

                def node(n, carry2):
                    rbase = n * Sp
                    for k in range(D // 16):
                        sl = pl.ds(k * 16, 16)
                        acc = rows[rbase, sl]
                        for j in range(1, S):
                            acc = acc + rows[rbase + j, sl]
                        if chunk_writes:
                            out_v[n, sl] = acc
                        else:
                            out_v[t * C + n, sl] = acc
                    return carry2

                lax.fori_loop(0, C, node, 0)
                if chunk_writes:
                    pltpu.sync_copy(
                        out_v, out_hbm.at[pl.ds(wid * R + t * C, C)]
                    )

            fire(0, 0)

            def pair(u, carry):
                t0 = u * 2
                fire(t0 + 1, 1)
                drain(0)
                compute(t0, 0)
                pl.when(t0 + 2 < n_chunks)(lambda: fire(t0 + 2, 0))
                drain(1)
                compute(t0 + 1, 1)
                return carry

            lax.fori_loop(0, n_chunks // 2, pair, 0)
            if not chunk_writes:
                pltpu.sync_copy(out_v, out_hbm.at[pl.ds(wid * R, R)])

        if n_active < NW:
            pl.when(wid < n_active)(work)
        else:
            work()

    return seg_sum


def _mm_relu_body(h_ref, s_ref, wa_ref, wb_ref, o_ref):
    o_ref[...] = jnp.maximum(
        jnp.dot(h_ref[...], wa_ref[...], preferred_element_type=jnp.float32)
        + jnp.dot(s_ref[...], wb_ref[...], preferred_element_type=jnp.float32),
        0.0,
    )


def _mm_relu(h, s, wa, wb, block_rows=512):
    n = h.shape[0]
    grid = n // block_rows
    return pl.pallas_call(
        _mm_relu_body,
        grid=(grid,),
        in_specs=[
            pl.BlockSpec((block_rows, D), lambda i: (i, 0)),
            pl.BlockSpec((block_rows, D), lambda i: (i, 0)),
            pl.BlockSpec((D, D), lambda i: (0, 0)),
            pl.BlockSpec((D, D), lambda i: (0, 0)),
        ],
        out_specs=pl.BlockSpec((block_rows, D), lambda i: (i, 0)),
        out_shape=jax.ShapeDtypeStruct((n, D), jnp.float32),
    )(h, s, wa, wb)


def _cls_body(p_ref, w_ref, b_ref, o_ref):
    logits = (
        jnp.dot(p_ref[...], w_ref[...], preferred_element_type=jnp.float32)
        + b_ref[0:1, :]
    )
    m = jnp.max(logits, axis=1, keepdims=True)
    lse = m + jnp.log(jnp.sum(jnp.exp(logits - m), axis=1, keepdims=True))
    o_ref[...] = logits - lse


def _classifier(p, wpad, bpad):
    return pl.pallas_call(
        _cls_body,
        grid=(1,),
        in_specs=[
            pl.BlockSpec((N_GRAPHS, D), lambda i: (0, 0)),
            pl.BlockSpec((D, D), lambda i: (0, 0)),
            pl.BlockSpec((8, D), lambda i: (0, 0)),
        ],
        out_specs=pl.BlockSpec((N_GRAPHS, D), lambda i: (0, 0)),
        out_shape=jax.ShapeDtypeStruct((N_GRAPHS, D), jnp.float32),
    )(p, wpad, bpad)


def kernel(target_samples, X, neigh_idx, W1, W2, W, b):
    NP = 10240

    Xp = jnp.pad(X, ((0, NP - N_NODES), (0, 0)))
    idx_flat = jnp.pad(
        neigh_idx.astype(jnp.int32).reshape(-1), (0, (NP - N_NODES) * M)
    ).reshape(-1, 128)

    layer_sum = _make_sc_segment_sum(n_out=NP, S=M, Sp=M, C=8)
    pool_sum = _make_sc_segment_sum(n_out=N_GRAPHS, S=NN, Sp=128, C=2, n_active=8)

    s1 = layer_sum(Xp, idx_flat)
    h1 = _mm_relu(Xp, s1, W1[:D], W1[D:] * (1.0 / M))
    s2 = layer_sum(h1, idx_flat)
    h2 = _mm_relu(h1, s2, W2[:D], W2[D:] * (1.0 / M))

    pool_idx = jnp.pad(target_samples.astype(jnp.int32), ((0, 0), (0, 128 - NN)))
    p = pool_sum(h2, pool_idx)

    n_cls = W.shape[1]
    wpad = jnp.zeros((D, D), jnp.float32).at[:, :n_cls].set(W * (1.0 / NN))
    bpad = jnp.full((8, D), -1e30, jnp.float32).at[:, :n_cls].set(b)
    out = _classifier(p, wpad, bpad)
    return out[:, :n_cls]


# speedup vs baseline: 4.2620x; 3.2594x over previous
import functools

import jax
import jax.numpy as jnp
from jax import lax
from jax.experimental import pallas as pl
from jax.experimental.pallas import tpu as pltpu
from jax.experimental.pallas import tpu_sc as plsc

N_NODES = 10000
M = 32
D = 128
NN = 100
N_GRAPHS = 64


def _make_sc_segment_sum(n_out, S, Sp, C, n_active=None, shared_table=None):
    info = plsc.get_sparse_core_info()
    NC, NS = info.num_cores, info.num_subcores
    NW = NC * NS
    if n_active is None:
        n_active = NW
    assert n_out % n_active == 0
    R = n_out // n_active
    assert R % C == 0 and R % 8 == 0
    n_chunks = R // C
    G = (C * Sp) // 128
    assert C * Sp % 128 == 0
    n_grp = (n_out * Sp) // 128
    grp_per_worker = (R * Sp) // 128
    whole_idx = (grp_per_worker % 8) != 0
    idx_rows = n_grp if whole_idx else grp_per_worker

    assert n_chunks % 2 == 0
    pair_rows = 2 * C
    assert pair_rows % 8 == 0 or n_active < NW
    chunk_writes = pair_rows % 8 == 0
    out_rows = pair_rows if chunk_writes else R
    mesh = plsc.VectorSubcoreMesh(core_axis_name="c", subcore_axis_name="s")

    scratch = [
        pltpu.VMEM((idx_rows, 128), jnp.int32),
        pltpu.VMEM((C * Sp, D), jnp.float32),
        pltpu.VMEM((C * Sp, D), jnp.float32),
        pltpu.VMEM((out_rows, D), jnp.float32),
        pltpu.SemaphoreType.DMA,
        pltpu.SemaphoreType.DMA,
    ]
    if shared_table is not None:
        assert shared_table % NS == 0
        stage_rows = shared_table // NS
        scratch.append(pltpu.VMEM_SHARED((shared_table, D), jnp.float32))

    @functools.partial(
        pl.kernel,
        mesh=mesh,
        out_type=jax.ShapeDtypeStruct((n_out, D), jnp.float32),
        scratch_types=scratch,
    )
    def seg_sum(table_hbm, idx_hbm, out_hbm, idx_v, rows_a, rows_b, out_v,
                sem_a, sem_b, *maybe_shared):
        cid = lax.axis_index("c")
        sid = lax.axis_index("s")
        wid = sid * NC + cid
        bufs = ((rows_a, sem_a), (rows_b, sem_b))

        if shared_table is not None:
            table = maybe_shared[0]
            pltpu.sync_copy(
                table_hbm.at[pl.ds(sid * stage_rows, stage_rows)],
                table.at[pl.ds(sid * stage_rows, stage_rows)],
            )
            plsc.subcore_barrier()
        else:
            table = table_hbm

        def work():
            if whole_idx:
                pltpu.sync_copy(idx_hbm, idx_v)
                grp_off = wid * grp_per_worker
            else:
                pltpu.sync_copy(
                    idx_hbm.at[pl.ds(wid * grp_per_worker, grp_per_worker)],
                    idx_v,
                )
                grp_off = 0

            def fire(t, buf):
                rows, sem = bufs[buf]
                for g in range(G):
                    pltpu.async_copy(
                        table.at[idx_v.at[grp_off + t * G + g]],
                        rows.at[pl.ds(g * 128, 128)],
                        sem,
                    )

            def drain(buf):
                rows, sem = bufs[buf]
                for g in range(G):
                    pltpu.make_async_copy(
                        table.at[idx_v.at[grp_off + g]],
                        rows.at[pl.ds(g * 128, 128)],
                        sem,
                    ).wait()

            def compute(t, buf, off):
                rows, _ = bufs[buf]

                def node(n, carry2):
                    rbase = n * Sp
                    for k in range(D // 16):
                        sl = pl.ds(k * 16, 16)
                        acc = rows[rbase, sl]
                        for j in range(1, S):
                            acc = acc + rows[rbase + j, sl]
                        if chunk_writes:
                            out_v[off + n, sl] = acc
                        else:
                            out_v[t * C + n, sl] = acc
                    return carry2

                lax.fori_loop(0, C, node, 0)

            fire(0, 0)

            def pair(u, carry):
                t0 = u * 2
                fire(t0 + 1, 1)
                drain(0)
                compute(t0, 0, 0)
                pl.when(t0 + 2 < n_chunks)(lambda: fire(t0 + 2, 0))
                drain(1)
                compute(t0 + 1, 1, C)
                if chunk_writes:
                    pltpu.sync_copy(
                        out_v,
                        out_hbm.at[pl.ds(wid * R + u * pair_rows, pair_rows)],
                    )
                return carry

            lax.fori_loop(0, n_chunks // 2, pair, 0)
            if not chunk_writes:
                pltpu.sync_copy(out_v, out_hbm.at[pl.ds(wid * R, R)])

        if n_active < NW:
            pl.when(wid < n_active)(work)
        else:
            work()

    return seg_sum


def _mm_relu_body(h_ref, s_ref, wa_ref, wb_ref, o_ref):
    o_ref[...] = jnp.maximum(
        jnp.dot(h_ref[...], wa_ref[...], preferred_element_type=jnp.float32)
        + jnp.dot(s_ref[...], wb_ref[...], preferred_element_type=jnp.float32),
        0.0,
    )


def _mm_relu(h, s, wa, wb, block_rows=512):
    n = h.shape[0]
    grid = n // block_rows
    return pl.pallas_call(
        _mm_relu_body,
        grid=(grid,),
        in_specs=[
            pl.BlockSpec((block_rows, D), lambda i: (i, 0)),
            pl.BlockSpec((block_rows, D), lambda i: (i, 0)),
            pl.BlockSpec((D, D), lambda i: (0, 0)),
            pl.BlockSpec((D, D), lambda i: (0, 0)),
        ],
        out_specs=pl.BlockSpec((block_rows, D), lambda i: (i, 0)),
        out_shape=jax.ShapeDtypeStruct((n, D), jnp.float32),
    )(h, s, wa, wb)


def _cls_body(p_ref, w_ref, b_ref, o_ref):
    logits = (
        jnp.dot(p_ref[...], w_ref[...], preferred_element_type=jnp.float32)
        + b_ref[0:1, :]
    )
    m = jnp.max(logits, axis=1, keepdims=True)
    lse = m + jnp.log(jnp.sum(jnp.exp(logits - m), axis=1, keepdims=True))
    o_ref[...] = logits - lse


def _classifier(p, wpad, bpad):
    return pl.pallas_call(
        _cls_body,
        grid=(1,),
        in_specs=[
            pl.BlockSpec((N_GRAPHS, D), lambda i: (0, 0)),
            pl.BlockSpec((D, D), lambda i: (0, 0)),
            pl.BlockSpec((8, D), lambda i: (0, 0)),
        ],
        out_specs=pl.BlockSpec((N_GRAPHS, D), lambda i: (0, 0)),
        out_shape=jax.ShapeDtypeStruct((N_GRAPHS, D), jnp.float32),
    )(p, wpad, bpad)


def kernel(target_samples, X, neigh_idx, W1, W2, W, b):
    NP = 10240

    Xp = jnp.pad(X, ((0, NP - N_NODES), (0, 0)))
    idx_flat = jnp.pad(
        neigh_idx.astype(jnp.int32).reshape(-1), (0, (NP - N_NODES) * M)
    ).reshape(-1, 128)

    layer_sum = _make_sc_segment_sum(n_out=NP, S=M, Sp=M, C=4, shared_table=NP)
    pool_sum = _make_sc_segment_sum(n_out=N_GRAPHS, S=NN, Sp=128, C=2, n_active=8)

    s1 = layer_sum(Xp, idx_flat)
    h1 = _mm_relu(Xp, s1, W1[:D], W1[D:] * (1.0 / M))
    s2 = layer_sum(h1, idx_flat)
    h2 = _mm_relu(h1, s2, W2[:D], W2[D:] * (1.0 / M))

    pool_idx = jnp.pad(target_samples.astype(jnp.int32), ((0, 0), (0, 128 - NN)))
    p = pool_sum(h2, pool_idx)

    n_cls = W.shape[1]
    wpad = jnp.zeros((D, D), jnp.float32).at[:, :n_cls].set(W * (1.0 / NN))
    bpad = jnp.full((8, D), -1e30, jnp.float32).at[:, :n_cls].set(b)
    out = _classifier(p, wpad, bpad)
    return out[:, :n_cls]


# speedup vs baseline: 5.0068x; 1.1747x over previous
import functools

import jax
import jax.numpy as jnp
from jax import lax
from jax.experimental import pallas as pl
from jax.experimental.pallas import tpu as pltpu
from jax.experimental.pallas import tpu_sc as plsc

N_NODES = 10000
M = 32
D = 128
NN = 100
N_GRAPHS = 64


def _make_sc_segment_sum(n_out, S, Sp, C, n_active=None, shared_table=None):
    info = plsc.get_sparse_core_info()
    NC, NS = info.num_cores, info.num_subcores
    NW = NC * NS
    if n_active is None:
        n_active = NW
    assert n_out % n_active == 0
    R = n_out // n_active
    assert R % C == 0 and R % 8 == 0
    n_chunks = R // C
    G = (C * Sp) // 128
    assert C * Sp % 128 == 0
    n_grp = (n_out * Sp) // 128
    grp_per_worker = (R * Sp) // 128
    whole_idx = (grp_per_worker % 8) != 0
    idx_rows = n_grp if whole_idx else grp_per_worker

    assert n_chunks % 2 == 0
    pair_rows = 2 * C
    assert pair_rows % 8 == 0 or n_active < NW
    chunk_writes = pair_rows % 8 == 0
    out_rows = pair_rows if chunk_writes else R
    mesh = plsc.VectorSubcoreMesh(core_axis_name="c", subcore_axis_name="s")

    scratch = [
        pltpu.VMEM((idx_rows, 128), jnp.int32),
        pltpu.VMEM((C * Sp, D), jnp.float32),
        pltpu.VMEM((C * Sp, D), jnp.float32),
        pltpu.VMEM((out_rows, D), jnp.float32),
        pltpu.SemaphoreType.DMA,
        pltpu.SemaphoreType.DMA,
    ]
    if shared_table is not None:
        assert shared_table % NS == 0
        stage_rows = shared_table // NS
        scratch.append(pltpu.VMEM_SHARED((shared_table, D), jnp.float32))

    @functools.partial(
        pl.kernel,
        mesh=mesh,
        out_type=jax.ShapeDtypeStruct((n_out, D), jnp.float32),
        scratch_types=scratch,
    )
    def seg_sum(table_hbm, idx_hbm, out_hbm, idx_v, rows_a, rows_b, out_v,
                sem_a, sem_b, *maybe_shared):
        cid = lax.axis_index("c")
        sid = lax.axis_index("s")
        wid = sid * NC + cid
        bufs = ((rows_a, sem_a), (rows_b, sem_b))

        if shared_table is not None:
            table = maybe_shared[0]
            pltpu.sync_copy(
                table_hbm.at[pl.ds(sid * stage_rows, stage_rows)],
                table.at[pl.ds(sid * stage_rows, stage_rows)],
            )
            plsc.subcore_barrier()
        else:
            table = table_hbm

        def work():
            if whole_idx:
                pltpu.sync_copy(idx_hbm, idx_v)
                grp_off = wid * grp_per_worker
            else:
                pltpu.sync_copy(
                    idx_hbm.at[pl.ds(wid * grp_per_worker, grp_per_worker)],
                    idx_v,
                )
                grp_off = 0

            def fire(t, buf):
                rows, sem = bufs[buf]
                for g in range(G):
                    pltpu.async_copy(
                        table.at[idx_v.at[grp_off + t * G + g]],
                        rows.at[pl.ds(g * 128, 128)],
                        sem,
                    )

            def drain(buf):
                rows, sem = bufs[buf]
                for g in range(G):
                    pltpu.make_async_copy(
                        table.at[idx_v.at[grp_off + g]],
                        rows.at[pl.ds(g * 128, 128)],
                        sem,
                    ).wait()

            def compute(t, buf, off):
                rows, _ = bufs[buf]

                def node(n, carry2):
                    rbase = n * Sp
                    for k in range(D // 16):
                        sl = pl.ds(k * 16, 16)
                        acc = rows[rbase, sl]
                        for j in range(1, S):
                            acc = acc + rows[rbase + j, sl]
                        if chunk_writes:
                            out_v[off + n, sl] = acc
                        else:
                            out_v[t * C + n, sl] = acc
                    return carry2

                lax.fori_loop(0, C, node, 0)

            fire(0, 0)

            def pair(u, carry):
                t0 = u * 2
                fire(t0 + 1, 1)
                drain(0)
                compute(t0, 0, 0)
                pl.when(t0 + 2 < n_chunks)(lambda: fire(t0 + 2, 0))
                drain(1)
                compute(t0 + 1, 1, C)
                if chunk_writes:
                    pltpu.sync_copy(
                        out_v,
                        out_hbm.at[pl.ds(wid * R + u * pair_rows, pair_rows)],
                    )
                return carry

            lax.fori_loop(0, n_chunks // 2, pair, 0)
            if not chunk_writes:
                pltpu.sync_copy(out_v, out_hbm.at[pl.ds(wid * R, R)])

        if n_active < NW:
            pl.when(wid < n_active)(work)
        else:
            work()

    return seg_sum


def _mm_relu_body(h_ref, s_ref, wa_ref, wb_ref, o_ref):
    o_ref[...] = jnp.maximum(
        jnp.dot(h_ref[...], wa_ref[...], preferred_element_type=jnp.float32)
        + jnp.dot(s_ref[...], wb_ref[...], preferred_element_type=jnp.float32),
        0.0,
    )


def _mm_relu(h, s, wa, wb, block_rows=512):
    n = h.shape[0]
    grid = n // block_rows
    return pl.pallas_call(
        _mm_relu_body,
        grid=(grid,),
        in_specs=[
            pl.BlockSpec((block_rows, D), lambda i: (i, 0)),
            pl.BlockSpec((block_rows, D), lambda i: (i, 0)),
            pl.BlockSpec((D, D), lambda i: (0, 0)),
            pl.BlockSpec((D, D), lambda i: (0, 0)),
        ],
        out_specs=pl.BlockSpec((block_rows, D), lambda i: (i, 0)),
        out_shape=jax.ShapeDtypeStruct((n, D), jnp.float32),
    )(h, s, wa, wb)


def _cls_body(p_ref, w_ref, b_ref, o_ref):
    logits = (
        jnp.dot(p_ref[...], w_ref[...], preferred_element_type=jnp.float32)
        + b_ref[0:1, :]
    )
    m = jnp.max(logits, axis=1, keepdims=True)
    lse = m + jnp.log(jnp.sum(jnp.exp(logits - m), axis=1, keepdims=True))
    o_ref[...] = logits - lse


def _classifier(p, wpad, bpad):
    return pl.pallas_call(
        _cls_body,
        grid=(1,),
        in_specs=[
            pl.BlockSpec((N_GRAPHS, D), lambda i: (0, 0)),
            pl.BlockSpec((D, D), lambda i: (0, 0)),
            pl.BlockSpec((8, D), lambda i: (0, 0)),
        ],
        out_specs=pl.BlockSpec((N_GRAPHS, D), lambda i: (0, 0)),
        out_shape=jax.ShapeDtypeStruct((N_GRAPHS, D), jnp.float32),
    )(p, wpad, bpad)


def kernel(target_samples, X, neigh_idx, W1, W2, W, b):
    NP = 10240

    Xp = jnp.pad(X, ((0, NP - N_NODES), (0, 0)))
    idx_flat = jnp.pad(
        neigh_idx.astype(jnp.int32).reshape(-1), (0, (NP - N_NODES) * M)
    ).reshape(-1, 128)

    layer_sum = _make_sc_segment_sum(n_out=NP, S=M, Sp=M, C=4, shared_table=NP)
    pool_sum = _make_sc_segment_sum(n_out=N_GRAPHS, S=NN, Sp=128, C=1,
                                    n_active=8, shared_table=NP)

    s1 = layer_sum(Xp, idx_flat)
    h1 = _mm_relu(Xp, s1, W1[:D], W1[D:] * (1.0 / M))
    s2 = layer_sum(h1, idx_flat)
    h2 = _mm_relu(h1, s2, W2[:D], W2[D:] * (1.0 / M))

    pool_idx = jnp.pad(target_samples.astype(jnp.int32), ((0, 0), (0, 128 - NN)))
    p = pool_sum(h2, pool_idx)

    n_cls = W.shape[1]
    wpad = jnp.zeros((D, D), jnp.float32).at[:, :n_cls].set(W * (1.0 / NN))
    bpad = jnp.full((8, D), -1e30, jnp.float32).at[:, :n_cls].set(b)
    out = _classifier(p, wpad, bpad)
    return out[:, :n_cls]


# speedup vs baseline: 5.6284x; 1.1241x over previous
import functools

import jax
import jax.numpy as jnp
from jax import lax
from jax.experimental import pallas as pl
from jax.experimental.pallas import tpu as pltpu
from jax.experimental.pallas import tpu_sc as plsc

N_NODES = 10000
M = 32
D = 128
NN = 100
N_GRAPHS = 64


def _make_sc_segment_sum(n_out, S, Sp, C, n_active=None, shared_table=None):
    info = plsc.get_sparse_core_info()
    NC, NS = info.num_cores, info.num_subcores
    NW = NC * NS
    if n_active is None:
        n_active = NW
    assert n_out % n_active == 0
    R = n_out // n_active
    assert R % C == 0 and R % 8 == 0
    n_chunks = R // C
    G = (C * Sp) // 128
    assert C * Sp % 128 == 0
    n_grp = (n_out * Sp) // 128
    grp_per_worker = (R * Sp) // 128
    whole_idx = (grp_per_worker % 8) != 0
    idx_rows = n_grp if whole_idx else grp_per_worker

    assert n_chunks % 2 == 0
    pair_rows = 2 * C
    assert pair_rows % 8 == 0 or n_active < NW
    chunk_writes = pair_rows % 8 == 0
    out_rows = pair_rows if chunk_writes else R
    mesh = plsc.VectorSubcoreMesh(core_axis_name="c", subcore_axis_name="s")

    scratch = [
        pltpu.VMEM((idx_rows, 128), jnp.int32),
        pltpu.VMEM((C * Sp, D), jnp.float32),
        pltpu.VMEM((C * Sp, D), jnp.float32),
        pltpu.VMEM((out_rows, D), jnp.float32),
        pltpu.SemaphoreType.DMA,
        pltpu.SemaphoreType.DMA,
    ]
    if shared_table is not None:
        assert shared_table % NS == 0
        stage_rows = shared_table // NS
        scratch.append(pltpu.VMEM_SHARED((shared_table, D), jnp.float32))

    @functools.partial(
        pl.kernel,
        mesh=mesh,
        out_type=jax.ShapeDtypeStruct((n_out, D), jnp.float32),
        scratch_types=scratch,
    )
    def seg_sum(table_hbm, idx_hbm, out_hbm, idx_v, rows_a, rows_b, out_v,
                sem_a, sem_b, *maybe_shared):
        cid = lax.axis_index("c")
        sid = lax.axis_index("s")
        wid = sid * NC + cid
        bufs = ((rows_a, sem_a), (rows_b, sem_b))

        if shared_table is not None:
            table = maybe_shared[0]
            pltpu.sync_copy(
                table_hbm.at[pl.ds(sid * stage_rows, stage_rows)],
                table.at[pl.ds(sid * stage_rows, stage_rows)],
            )
            plsc.subcore_barrier()
        else:
            table = table_hbm

        def work():
            if whole_idx:
                pltpu.sync_copy(idx_hbm, idx_v)
                grp_off = wid * grp_per_worker
            else:
                pltpu.sync_copy(
                    idx_hbm.at[pl.ds(wid * grp_per_worker, grp_per_worker)],
                    idx_v,
                )
                grp_off = 0

            def fire(t, buf):
                rows, sem = bufs[buf]
                for g in range(G):
                    pltpu.async_copy(
                        table.at[idx_v.at[grp_off + t * G + g]],
                        rows.at[pl.ds(g * 128, 128)],
                        sem,
                    )

            def drain(buf):
                rows, sem = bufs[buf]
                for g in range(G):
                    pltpu.make_async_copy(
                        table.at[idx_v.at[grp_off + g]],
                        rows.at[pl.ds(g * 128, 128)],
                        sem,
                    ).wait()

            def compute(t, buf, off):
                rows, _ = bufs[buf]

                def node(n, carry2):
                    rbase = n * Sp
                    for k in range(D // 16):
                        sl = pl.ds(k * 16, 16)
                        lanes = min(4, S)
                        accs = [rows[rbase + j, sl] for j in range(lanes)]
                        for j in range(lanes, S):
                            accs[j % lanes] = accs[j % lanes] + rows[rbase + j, sl]
                        acc = (accs[0] + accs[1]) + (accs[2] + accs[3]) \
                            if lanes == 4 else sum(accs[1:], accs[0])
                        if chunk_writes:
                            out_v[off + n, sl] = acc
                        else:
                            out_v[t * C + n, sl] = acc
                    return carry2

                lax.fori_loop(0, C, node, 0)

            fire(0, 0)

            def pair(u, carry):
                t0 = u * 2
                fire(t0 + 1, 1)
                drain(0)
                compute(t0, 0, 0)
                pl.when(t0 + 2 < n_chunks)(lambda: fire(t0 + 2, 0))
                drain(1)
                compute(t0 + 1, 1, C)
                if chunk_writes:
                    pltpu.sync_copy(
                        out_v,
                        out_hbm.at[pl.ds(wid * R + u * pair_rows, pair_rows)],
                    )
                return carry

            lax.fori_loop(0, n_chunks // 2, pair, 0)
            if not chunk_writes:
                pltpu.sync_copy(out_v, out_hbm.at[pl.ds(wid * R, R)])

        if n_active < NW:
            pl.when(wid < n_active)(work)
        else:
            work()

    return seg_sum


def _mm_relu_body(h_ref, s_ref, wa_ref, wb_ref, o_ref):
    o_ref[...] = jnp.maximum(
        jnp.dot(h_ref[...], wa_ref[...], preferred_element_type=jnp.float32)
        + jnp.dot(s_ref[...], wb_ref[...], preferred_element_type=jnp.float32),
        0.0,
    )


def _mm_relu(h, s, wa, wb, block_rows=512):
    n = h.shape[0]
    grid = n // block_rows
    return pl.pallas_call(
        _mm_relu_body,
        grid=(grid,),
        in_specs=[
            pl.BlockSpec((block_rows, D), lambda i: (i, 0)),
            pl.BlockSpec((block_rows, D), lambda i: (i, 0)),
            pl.BlockSpec((D, D), lambda i: (0, 0)),
            pl.BlockSpec((D, D), lambda i: (0, 0)),
        ],
        out_specs=pl.BlockSpec((block_rows, D), lambda i: (i, 0)),
        out_shape=jax.ShapeDtypeStruct((n, D), jnp.float32),
    )(h, s, wa, wb)


def _cls_body(p_ref, w_ref, b_ref, o_ref):
    logits = (
        jnp.dot(p_ref[...], w_ref[...], preferred_element_type=jnp.float32)
        + b_ref[0:1, :]
    )
    m = jnp.max(logits, axis=1, keepdims=True)
    lse = m + jnp.log(jnp.sum(jnp.exp(logits - m), axis=1, keepdims=True))
    o_ref[...] = logits - lse


def _classifier(p, wpad, bpad):
    return pl.pallas_call(
        _cls_body,
        grid=(1,),
        in_specs=[
            pl.BlockSpec((N_GRAPHS, D), lambda i: (0, 0)),
            pl.BlockSpec((D, D), lambda i: (0, 0)),
            pl.BlockSpec((8, D), lambda i: (0, 0)),
        ],
        out_specs=pl.BlockSpec((N_GRAPHS, D), lambda i: (0, 0)),
        out_shape=jax.ShapeDtypeStruct((N_GRAPHS, D), jnp.float32),
    )(p, wpad, bpad)


def kernel(target_samples, X, neigh_idx, W1, W2, W, b):
    NP = 10240

    Xp = jnp.pad(X, ((0, NP - N_NODES), (0, 0)))
    idx_flat = jnp.pad(
        neigh_idx.astype(jnp.int32).reshape(-1), (0, (NP - N_NODES) * M)
    ).reshape(-1, 128)

    layer_sum = _make_sc_segment_sum(n_out=NP, S=M, Sp=M, C=4, shared_table=NP)
    pool_sum = _make_sc_segment_sum(n_out=N_GRAPHS, S=NN, Sp=128, C=1,
                                    n_active=8, shared_table=NP)

    s1 = layer_sum(Xp, idx_flat)
    h1 = _mm_relu(Xp, s1, W1[:D], W1[D:] * (1.0 / M))
    s2 = layer_sum(h1, idx_flat)
    h2 = _mm_relu(h1, s2, W2[:D], W2[D:] * (1.0 / M))

    pool_idx = jnp.pad(target_samples.astype(jnp.int32), ((0, 0), (0, 128 - NN)))
    p = pool_sum(h2, pool_idx)

    n_cls = W.shape[1]
    wpad = jnp.zeros((D, D), jnp.float32).at[:, :n_cls].set(W * (1.0 / NN))
    bpad = jnp.full((8, D), -1e30, jnp.float32).at[:, :n_cls].set(b)
    out = _classifier(p, wpad, bpad)
    return out[:, :n_cls]


# speedup vs baseline: 7.4644x; 1.3262x over previous
import functools

import jax
import jax.numpy as jnp
from jax import lax
from jax.experimental import pallas as pl
from jax.experimental.pallas import tpu as pltpu
from jax.experimental.pallas import tpu_sc as plsc

N_NODES = 10000
M = 32
D = 128
NN = 100
N_GRAPHS = 64


def _make_sc_segment_sum(n_out, S, Sp, C, n_active=None, shared_table=None):
    info = plsc.get_sparse_core_info()
    NC, NS = info.num_cores, info.num_subcores
    NW = NC * NS
    if n_active is None:
        n_active = NW
    assert n_out % n_active == 0
    R = n_out // n_active
    assert R % C == 0 and R % 8 == 0
    n_chunks = R // C
    G = (C * Sp) // 128
    assert C * Sp % 128 == 0
    n_grp = (n_out * Sp) // 128
    grp_per_worker = (R * Sp) // 128
    whole_idx = (grp_per_worker % 8) != 0
    idx_rows = n_grp if whole_idx else grp_per_worker

    assert n_chunks % 2 == 0
    pair_rows = 2 * C
    assert pair_rows % 8 == 0 or n_active < NW
    chunk_writes = pair_rows % 8 == 0
    out_rows = pair_rows if chunk_writes else R
    mesh = plsc.VectorSubcoreMesh(core_axis_name="c", subcore_axis_name="s")

    scratch = [
        pltpu.VMEM((idx_rows, 128), jnp.int32),
        pltpu.VMEM((C * Sp, D), jnp.float32),
        pltpu.VMEM((C * Sp, D), jnp.float32),
        pltpu.VMEM((out_rows, D), jnp.float32),
        pltpu.SemaphoreType.DMA,
        pltpu.SemaphoreType.DMA,
    ]
    if shared_table is not None:
        assert shared_table % NS == 0
        stage_rows = shared_table // NS
        scratch.append(pltpu.VMEM_SHARED((shared_table, D), jnp.float32))

    @functools.partial(
        pl.kernel,
        mesh=mesh,
        out_type=jax.ShapeDtypeStruct((n_out, D), jnp.float32),
        scratch_types=scratch,
    )
    def seg_sum(table_hbm, idx_hbm, out_hbm, idx_v, rows_a, rows_b, out_v,
                sem_a, sem_b, *maybe_shared):
        cid = lax.axis_index("c")
        sid = lax.axis_index("s")
        wid = sid * NC + cid
        bufs = ((rows_a, sem_a), (rows_b, sem_b))

        if shared_table is not None:
            table = maybe_shared[0]
            pltpu.sync_copy(
                table_hbm.at[pl.ds(sid * stage_rows, stage_rows)],
                table.at[pl.ds(sid * stage_rows, stage_rows)],
            )
            plsc.subcore_barrier()
        else:
            table = table_hbm

        def work():
            if whole_idx:
                pltpu.sync_copy(idx_hbm, idx_v)
                grp_off = wid * grp_per_worker
            else:
                pltpu.sync_copy(
                    idx_hbm.at[pl.ds(wid * grp_per_worker, grp_per_worker)],
                    idx_v,
                )
                grp_off = 0

            def fire(t, buf):
                rows, sem = bufs[buf]
                for g in range(G):
                    pltpu.async_copy(
                        table.at[idx_v.at[grp_off + t * G + g]],
                        rows.at[pl.ds(g * 128, 128)],
                        sem,
                    )

            def drain(buf):
                rows, sem = bufs[buf]
                for g in range(G):
                    pltpu.make_async_copy(
                        table.at[idx_v.at[grp_off + g]],
                        rows.at[pl.ds(g * 128, 128)],
                        sem,
                    ).wait()

            def compute(t, buf, off):
                rows, _ = bufs[buf]

                def node(n, carry2):
                    rbase = n * Sp
                    for k in range(D // 16):
                        sl = pl.ds(k * 16, 16)
                        lanes = min(4, S)
                        accs = [rows[rbase + j, sl] for j in range(lanes)]
                        for j in range(lanes, S):
                            accs[j % lanes] = accs[j % lanes] + rows[rbase + j, sl]
                        acc = (accs[0] + accs[1]) + (accs[2] + accs[3]) \
                            if lanes == 4 else sum(accs[1:], accs[0])
                        if chunk_writes:
                            out_v[off + n, sl] = acc
                        else:
                            out_v[t * C + n, sl] = acc
                    return carry2

                lax.fori_loop(0, C, node, 0)

            fire(0, 0)

            def pair(u, carry):
                t0 = u * 2
                fire(t0 + 1, 1)
                drain(0)
                compute(t0, 0, 0)
                pl.when(t0 + 2 < n_chunks)(lambda: fire(t0 + 2, 0))
                drain(1)
                compute(t0 + 1, 1, C)
                if chunk_writes:
                    pltpu.sync_copy(
                        out_v,
                        out_hbm.at[pl.ds(wid * R + u * pair_rows, pair_rows)],
                    )
                return carry

            lax.fori_loop(0, n_chunks // 2, pair, 0)
            if not chunk_writes:
                pltpu.sync_copy(out_v, out_hbm.at[pl.ds(wid * R, R)])

        if n_active < NW:
            pl.when(wid < n_active)(work)
        else:
            work()

    return seg_sum


def _make_sc_gather_add(n_out, S, Cg, shared_table):
    info = plsc.get_sparse_core_info()
    NC, NS = info.num_cores, info.num_subcores
    NW = NC * NS
    assert n_out % NW == 0
    R = n_out // NW
    assert R % Cg == 0 and Cg % 8 == 0
    n_chunks = R // Cg
    assert n_chunks % 2 == 0
    rows_per_worker = n_chunks * S
    assert (rows_per_worker % 8) == 0
    assert shared_table % NS == 0
    stage_rows = shared_table // NS

    mesh = plsc.VectorSubcoreMesh(core_axis_name="c", subcore_axis_name="s")

    @functools.partial(
        pl.kernel,
        mesh=mesh,
        out_type=jax.ShapeDtypeStruct((n_out, D), jnp.float32),
        scratch_types=[
            pltpu.VMEM((rows_per_worker, Cg), jnp.int32),
            pltpu.VMEM((Cg, D), jnp.float32),
            pltpu.VMEM((Cg, D), jnp.float32),
            pltpu.SemaphoreType.DMA,
            pltpu.SemaphoreType.DMA,
            pltpu.VMEM_SHARED((shared_table, D), jnp.float32),
        ],
    )
    def seg_sum(table_hbm, idx_hbm, out_hbm, idx_v, acc_a, acc_b,
                sem_a, sem_b, table):
        cid = lax.axis_index("c")
        sid = lax.axis_index("s")
        wid = sid * NC + cid
        bufs = ((acc_a, sem_a), (acc_b, sem_b))

        pltpu.sync_copy(
            table_hbm.at[pl.ds(sid * stage_rows, stage_rows)],
            table.at[pl.ds(sid * stage_rows, stage_rows)],
        )
        plsc.subcore_barrier()

        pltpu.sync_copy(
            idx_hbm.at[pl.ds(wid * rows_per_worker, rows_per_worker)], idx_v
        )

        zval = jnp.zeros((16,), jnp.float32)

        def zero(buf):
            acc, _ = bufs[buf]

            def zrow(n, carry):
                for k in range(D // 16):
                    acc[n, pl.ds(k * 16, 16)] = zval
                return carry

            lax.fori_loop(0, Cg, zrow, 0)

        def fire(t, buf):
            acc, sem = bufs[buf]
            for j in range(S):
                pltpu.async_copy(
                    table.at[idx_v.at[t * S + j]], acc, sem, add=True
                )

        def drain(buf):
            acc, sem = bufs[buf]
            for j in range(S):
                pltpu.make_async_copy(
                    table.at[idx_v.at[j]], acc, sem
                ).wait()

        def flush(t, buf):
            acc, _ = bufs[buf]
            pltpu.sync_copy(acc, out_hbm.at[pl.ds(wid * R + t * Cg, Cg)])

        zero(0)
        zero(1)
        fire(0, 0)
        fire(1, 1)

        def pair(u, carry):
            t0 = u * 2
            drain(0)
            flush(t0, 0)
            zero(0)
            pl.when(t0 + 2 < n_chunks)(lambda: fire(t0 + 2, 0))
            drain(1)
            flush(t0 + 1, 1)
            zero(1)
            pl.when(t0 + 3 < n_chunks)(lambda: fire(t0 + 3, 1))
            return carry

        lax.fori_loop(0, n_chunks // 2, pair, 0)

    return seg_sum


def _mm_relu_body(h_ref, s_ref, wa_ref, wb_ref, o_ref):
    o_ref[...] = jnp.maximum(
        jnp.dot(h_ref[...], wa_ref[...], preferred_element_type=jnp.float32)
        + jnp.dot(s_ref[...], wb_ref[...], preferred_element_type=jnp.float32),
        0.0,
    )


def _mm_relu(h, s, wa, wb, block_rows=512):
    n = h.shape[0]
    grid = n // block_rows
    return pl.pallas_call(
        _mm_relu_body,
        grid=(grid,),
        in_specs=[
            pl.BlockSpec((block_rows, D), lambda i: (i, 0)),
            pl.BlockSpec((block_rows, D), lambda i: (i, 0)),
            pl.BlockSpec((D, D), lambda i: (0, 0)),
            pl.BlockSpec((D, D), lambda i: (0, 0)),
        ],
        out_specs=pl.BlockSpec((block_rows, D), lambda i: (i, 0)),
        out_shape=jax.ShapeDtypeStruct((n, D), jnp.float32),
    )(h, s, wa, wb)


def _cls_body(p_ref, w_ref, b_ref, o_ref):
    logits = (
        jnp.dot(p_ref[...], w_ref[...], preferred_element_type=jnp.float32)
        + b_ref[0:1, :]
    )
    m = jnp.max(logits, axis=1, keepdims=True)
    lse = m + jnp.log(jnp.sum(jnp.exp(logits - m), axis=1, keepdims=True))
    o_ref[...] = logits - lse


def _classifier(p, wpad, bpad):
    return pl.pallas_call(
        _cls_body,
        grid=(1,),
        in_specs=[
            pl.BlockSpec((N_GRAPHS, D), lambda i: (0, 0)),
            pl.BlockSpec((D, D), lambda i: (0, 0)),
            pl.BlockSpec((8, D), lambda i: (0, 0)),
        ],
        out_specs=pl.BlockSpec((N_GRAPHS, D), lambda i: (0, 0)),
        out_shape=jax.ShapeDtypeStruct((N_GRAPHS, D), jnp.float32),
    )(p, wpad, bpad)


def kernel(target_samples, X, neigh_idx, W1, W2, W, b):
    NP = 10240

    CG = 32

    Xp = jnp.pad(X, ((0, NP - N_NODES), (0, 0)))
    idx_t = (
        jnp.pad(neigh_idx.astype(jnp.int32), ((0, NP - N_NODES), (0, 0)))
        .reshape(NP // CG, CG, M)
        .transpose(0, 2, 1)
        .reshape(-1, CG)
    )

    layer_sum = _make_sc_gather_add(n_out=NP, S=M, Cg=CG, shared_table=NP)
    pool_sum = _make_sc_segment_sum(n_out=N_GRAPHS, S=NN, Sp=128, C=1,
                                    n_active=8, shared_table=NP)

    s1 = layer_sum(Xp, idx_t)
    h1 = _mm_relu(Xp, s1, W1[:D], W1[D:] * (1.0 / M))
    s2 = layer_sum(h1, idx_t)
    h2 = _mm_relu(h1, s2, W2[:D], W2[D:] * (1.0 / M))

    pool_idx = jnp.pad(target_samples.astype(jnp.int32), ((0, 0), (0, 128 - NN)))
    p = pool_sum(h2, pool_idx)

    n_cls = W.shape[1]
    wpad = jnp.zeros((D, D), jnp.float32).at[:, :n_cls].set(W * (1.0 / NN))
    bpad = jnp.full((8, D), -1e30, jnp.float32).at[:, :n_cls].set(b)
    out = _classifier(p, wpad, bpad)
    return out[:, :n_cls]


# speedup vs baseline: 7.5436x; 1.0106x over previous
import functools

import jax
import jax.numpy as jnp
from jax import lax
from jax.experimental import pallas as pl
from jax.experimental.pallas import tpu as pltpu
from jax.experimental.pallas import tpu_sc as plsc

N_NODES = 10000
M = 32
D = 128
NN = 100
N_GRAPHS = 64


def _make_sc_segment_sum(n_out, S, Sp, C, n_active=None, shared_table=None):
    info = plsc.get_sparse_core_info()
    NC, NS = info.num_cores, info.num_subcores
    NW = NC * NS
    if n_active is None:
        n_active = NW
    assert n_out % n_active == 0
    R = n_out // n_active
    assert R % C == 0 and R % 8 == 0
    n_chunks = R // C
    G = (C * Sp) // 128
    assert C * Sp % 128 == 0
    n_grp = (n_out * Sp) // 128
    grp_per_worker = (R * Sp) // 128
    whole_idx = (grp_per_worker % 8) != 0
    idx_rows = n_grp if whole_idx else grp_per_worker

    assert n_chunks % 2 == 0
    pair_rows = 2 * C
    assert pair_rows % 8 == 0 or n_active < NW
    chunk_writes = pair_rows % 8 == 0
    out_rows = pair_rows if chunk_writes else R
    mesh = plsc.VectorSubcoreMesh(core_axis_name="c", subcore_axis_name="s")

    scratch = [
        pltpu.VMEM((idx_rows, 128), jnp.int32),
        pltpu.VMEM((C * Sp, D), jnp.float32),
        pltpu.VMEM((C * Sp, D), jnp.float32),
        pltpu.VMEM((out_rows, D), jnp.float32),
        pltpu.SemaphoreType.DMA,
        pltpu.SemaphoreType.DMA,
    ]
    if shared_table is not None:
        assert shared_table % NS == 0
        stage_rows = shared_table // NS
        scratch.append(pltpu.VMEM_SHARED((shared_table, D), jnp.float32))

    @functools.partial(
        pl.kernel,
        mesh=mesh,
        out_type=jax.ShapeDtypeStruct((n_out, D), jnp.float32),
        scratch_types=scratch,
    )
    def seg_sum(table_hbm, idx_hbm, out_hbm, idx_v, rows_a, rows_b, out_v,
                sem_a, sem_b, *maybe_shared):
        cid = lax.axis_index("c")
        sid = lax.axis_index("s")
        wid = sid * NC + cid
        bufs = ((rows_a, sem_a), (rows_b, sem_b))

        if shared_table is not None:
            table = maybe_shared[0]
            pltpu.sync_copy(
                table_hbm.at[pl.ds(sid * stage_rows, stage_rows)],
                table.at[pl.ds(sid * stage_rows, stage_rows)],
            )
            plsc.subcore_barrier()
        else:
            table = table_hbm

        def work():
            if whole_idx:
                pltpu.sync_copy(idx_hbm, idx_v)
                grp_off = wid * grp_per_worker
            else:
                pltpu.sync_copy(
                    idx_hbm.at[pl.ds(wid * grp_per_worker, grp_per_worker)],
                    idx_v,
                )
                grp_off = 0

            def fire(t, buf):
                rows, sem = bufs[buf]
                for g in range(G):
                    pltpu.async_copy(
                        table.at[idx_v.at[grp_off + t * G + g]],
                        rows.at[pl.ds(g * 128, 128)],
                        sem,
                    )

            def drain(buf):
                rows, sem = bufs[buf]
                for g in range(G):
                    pltpu.make_async_copy(
                        table.at[idx_v.at[grp_off + g]],
                        rows.at[pl.ds(g * 128, 128)],
                        sem,
                    ).wait()

            def compute(t, buf, off):
                rows, _ = bufs[buf]

                def node(n, carry2):
                    rbase = n * Sp
                    for k in range(D // 16):
                        sl = pl.ds(k * 16, 16)
                        lanes = min(4, S)
                        accs = [rows[rbase + j, sl] for j in range(lanes)]
                        for j in range(lanes, S):
                            accs[j % lanes] = accs[j % lanes] + rows[rbase + j, sl]
                        acc = (accs[0] + accs[1]) + (accs[2] + accs[3]) \
                            if lanes == 4 else sum(accs[1:], accs[0])
                        if chunk_writes:
                            out_v[off + n, sl] = acc
                        else:
                            out_v[t * C + n, sl] = acc
                    return carry2

                lax.fori_loop(0, C, node, 0)

            fire(0, 0)

            def pair(u, carry):
                t0 = u * 2
                fire(t0 + 1, 1)
                drain(0)
                compute(t0, 0, 0)
                pl.when(t0 + 2 < n_chunks)(lambda: fire(t0 + 2, 0))
                drain(1)
                compute(t0 + 1, 1, C)
                if chunk_writes:
                    pltpu.sync_copy(
                        out_v,
                        out_hbm.at[pl.ds(wid * R + u * pair_rows, pair_rows)],
                    )
                return carry

            lax.fori_loop(0, n_chunks // 2, pair, 0)
            if not chunk_writes:
                pltpu.sync_copy(out_v, out_hbm.at[pl.ds(wid * R, R)])

        if n_active < NW:
            pl.when(wid < n_active)(work)
        else:
            work()

    return seg_sum


def _make_sc_gather_add(n_out, S, Cg, shared_table=None, n_active=None, d=D):
    info = plsc.get_sparse_core_info()
    NC, NS = info.num_cores, info.num_subcores
    NW = NC * NS
    if n_active is None:
        n_active = NW
    assert n_out % n_active == 0
    R = n_out // n_active
    assert R % Cg == 0 and Cg % 8 == 0
    n_chunks = R // Cg
    assert n_chunks % 2 == 0
    rows_per_worker = n_chunks * S
    assert (rows_per_worker % 8) == 0

    scratch = [
        pltpu.VMEM((rows_per_worker, Cg), jnp.int32),
        pltpu.VMEM((Cg, d), jnp.float32),
        pltpu.VMEM((Cg, d), jnp.float32),
        pltpu.SemaphoreType.DMA,
        pltpu.SemaphoreType.DMA,
    ]
    if shared_table is not None:
        assert shared_table % NS == 0
        stage_rows = shared_table // NS
        scratch.append(pltpu.VMEM_SHARED((shared_table, d), jnp.float32))

    mesh = plsc.VectorSubcoreMesh(core_axis_name="c", subcore_axis_name="s")

    @functools.partial(
        pl.kernel,
        mesh=mesh,
        out_type=jax.ShapeDtypeStruct((n_out, d), jnp.float32),
        scratch_types=scratch,
    )
    def seg_sum(table_hbm, idx_hbm, out_hbm, idx_v, acc_a, acc_b,
                sem_a, sem_b, *maybe_shared):
        cid = lax.axis_index("c")
        sid = lax.axis_index("s")
        wid = sid * NC + cid
        bufs = ((acc_a, sem_a), (acc_b, sem_b))

        if shared_table is not None:
            table = maybe_shared[0]
            pltpu.sync_copy(
                table_hbm.at[pl.ds(sid * stage_rows, stage_rows)],
                table.at[pl.ds(sid * stage_rows, stage_rows)],
            )
            plsc.subcore_barrier()
        else:
            table = table_hbm

        zval = jnp.zeros((16,), jnp.float32)

        def work():
            pltpu.sync_copy(
                idx_hbm.at[pl.ds(wid * rows_per_worker, rows_per_worker)],
                idx_v,
            )

            def zero(buf):
                acc, _ = bufs[buf]

                def zrow(n, carry):
                    for k in range(d // 16):
                        acc[n, pl.ds(k * 16, 16)] = zval
                    return carry

                lax.fori_loop(0, Cg, zrow, 0)

            def fire(t, buf):
                acc, sem = bufs[buf]
                for j in range(S):
                    pltpu.async_copy(
                        table.at[idx_v.at[t * S + j]], acc, sem, add=True
                    )

            def drain(buf):
                acc, sem = bufs[buf]
                for j in range(S):
                    pltpu.make_async_copy(
                        table.at[idx_v.at[j]], acc, sem
                    ).wait()

            def flush(t, buf):
                acc, _ = bufs[buf]
                pltpu.sync_copy(acc, out_hbm.at[pl.ds(wid * R + t * Cg, Cg)])

            zero(0)
            zero(1)
            fire(0, 0)
            fire(1, 1)

            def pair(u, carry):
                t0 = u * 2
                drain(0)
                flush(t0, 0)
                zero(0)
                pl.when(t0 + 2 < n_chunks)(lambda: fire(t0 + 2, 0))
                drain(1)
                flush(t0 + 1, 1)
                zero(1)
                pl.when(t0 + 3 < n_chunks)(lambda: fire(t0 + 3, 1))
                return carry

            lax.fori_loop(0, n_chunks // 2, pair, 0)

        if n_active < NW:
            pl.when(wid < n_active)(work)
        else:
            work()

    return seg_sum


def _mm_relu_body(h_ref, s_ref, wa_ref, wb_ref, o_ref):
    o_ref[...] = jnp.maximum(
        jnp.dot(h_ref[...], wa_ref[...], preferred_element_type=jnp.float32)
        + jnp.dot(s_ref[...], wb_ref[...], preferred_element_type=jnp.float32),
        0.0,
    )


def _mm_relu(h, s, wa, wb, block_rows=512):
    n = h.shape[0]
    grid = n // block_rows
    return pl.pallas_call(
        _mm_relu_body,
        grid=(grid,),
        in_specs=[
            pl.BlockSpec((block_rows, D), lambda i: (i, 0)),
            pl.BlockSpec((block_rows, D), lambda i: (i, 0)),
            pl.BlockSpec((D, D), lambda i: (0, 0)),
            pl.BlockSpec((D, D), lambda i: (0, 0)),
        ],
        out_specs=pl.BlockSpec((block_rows, D), lambda i: (i, 0)),
        out_shape=jax.ShapeDtypeStruct((n, D), jnp.float32),
    )(h, s, wa, wb)


DC = 128


def _mm_relu_cls_body(h_ref, s_ref, wa_ref, wb_ref, wc_ref, bc_ref, y_ref):
    h = jnp.maximum(
        jnp.dot(h_ref[...], wa_ref[...], preferred_element_type=jnp.float32)
        + jnp.dot(s_ref[...], wb_ref[...], preferred_element_type=jnp.float32),
        0.0,
    )
    y_ref[...] = (
        jnp.dot(h, wc_ref[...], preferred_element_type=jnp.float32)
        + bc_ref[0:1, :]
    )


def _mm_relu_cls(h, s, wa, wb, wc, bc, block_rows=512):
    n = h.shape[0]
    grid = n // block_rows
    return pl.pallas_call(
        _mm_relu_cls_body,
        grid=(grid,),
        in_specs=[
            pl.BlockSpec((block_rows, D), lambda i: (i, 0)),
            pl.BlockSpec((block_rows, D), lambda i: (i, 0)),
            pl.BlockSpec((D, D), lambda i: (0, 0)),
            pl.BlockSpec((D, D), lambda i: (0, 0)),
            pl.BlockSpec((D, DC), lambda i: (0, 0)),
            pl.BlockSpec((8, DC), lambda i: (0, 0)),
        ],
        out_specs=pl.BlockSpec((block_rows, DC), lambda i: (i, 0)),
        out_shape=jax.ShapeDtypeStruct((n, DC), jnp.float32),
    )(h, s, wa, wb, wc, bc)


def _ls_body(p_ref, o_ref):
    logits = p_ref[...] * (1.0 / NN)
    m = jnp.max(logits, axis=1, keepdims=True)
    lse = m + jnp.log(jnp.sum(jnp.exp(logits - m), axis=1, keepdims=True))
    o_ref[...] = logits - lse


def _log_softmax(p):
    return pl.pallas_call(
        _ls_body,
        grid=(1,),
        in_specs=[pl.BlockSpec((N_GRAPHS, DC), lambda i: (0, 0))],
        out_specs=pl.BlockSpec((N_GRAPHS, DC), lambda i: (0, 0)),
        out_shape=jax.ShapeDtypeStruct((N_GRAPHS, DC), jnp.float32),
    )(p)


def kernel(target_samples, X, neigh_idx, W1, W2, W, b):
    NP = 10240

    CG = 32

    Xp = jnp.pad(X, ((0, NP - N_NODES), (0, 0)))
    idx_t = (
        jnp.pad(neigh_idx.astype(jnp.int32), ((0, NP - N_NODES), (0, 0)))
        .reshape(NP // CG, CG, M)
        .transpose(0, 2, 1)
        .reshape(-1, CG)
    )

    layer_sum = _make_sc_gather_add(n_out=NP, S=M, Cg=CG, shared_table=NP)
    pool_sum = _make_sc_gather_add(n_out=N_GRAPHS, S=NN, Cg=8, n_active=4,
                                   d=DC)

    n_cls = W.shape[1]
    wc = jnp.zeros((D, DC), jnp.float32).at[:, :n_cls].set(W)
    bc = jnp.full((8, DC), -1e4, jnp.float32).at[:, :n_cls].set(b)

    s1 = layer_sum(Xp, idx_t)
    h1 = _mm_relu(Xp, s1, W1[:D], W1[D:] * (1.0 / M))
    s2 = layer_sum(h1, idx_t)
    y = _mm_relu_cls(h1, s2, W2[:D], W2[D:] * (1.0 / M), wc, bc)

    pool_idx = (
        target_samples.astype(jnp.int32)
        .reshape(N_GRAPHS // 8, 8, NN)
        .transpose(0, 2, 1)
        .reshape(-1, 8)
    )
    p = pool_sum(y, pool_idx)
    out = _log_softmax(p)
    return out[:, :n_cls]


# speedup vs baseline: 7.8332x; 1.0384x over previous
import functools

import jax
import jax.numpy as jnp
from jax import lax
from jax.experimental import pallas as pl
from jax.experimental.pallas import tpu as pltpu
from jax.experimental.pallas import tpu_sc as plsc

N_NODES = 10000
M = 32
D = 128
NN = 100
N_GRAPHS = 64


def _make_sc_segment_sum(n_out, S, Sp, C, n_active=None, shared_table=None):
    info = plsc.get_sparse_core_info()
    NC, NS = info.num_cores, info.num_subcores
    NW = NC * NS
    if n_active is None:
        n_active = NW
    assert n_out % n_active == 0
    R = n_out // n_active
    assert R % C == 0 and R % 8 == 0
    n_chunks = R // C
    G = (C * Sp) // 128
    assert C * Sp % 128 == 0
    n_grp = (n_out * Sp) // 128
    grp_per_worker = (R * Sp) // 128
    whole_idx = (grp_per_worker % 8) != 0
    idx_rows = n_grp if whole_idx else grp_per_worker

    assert n_chunks % 2 == 0
    pair_rows = 2 * C
    assert pair_rows % 8 == 0 or n_active < NW
    chunk_writes = pair_rows % 8 == 0
    out_rows = pair_rows if chunk_writes else R
    mesh = plsc.VectorSubcoreMesh(core_axis_name="c", subcore_axis_name="s")

    scratch = [
        pltpu.VMEM((idx_rows, 128), jnp.int32),
        pltpu.VMEM((C * Sp, D), jnp.float32),
        pltpu.VMEM((C * Sp, D), jnp.float32),
        pltpu.VMEM((out_rows, D), jnp.float32),
        pltpu.SemaphoreType.DMA,
        pltpu.SemaphoreType.DMA,
    ]
    if shared_table is not None:
        assert shared_table % NS == 0
        stage_rows = shared_table // NS
        scratch.append(pltpu.VMEM_SHARED((shared_table, D), jnp.float32))

    @functools.partial(
        pl.kernel,
        mesh=mesh,
        out_type=jax.ShapeDtypeStruct((n_out, D), jnp.float32),
        scratch_types=scratch,
    )
    def seg_sum(table_hbm, idx_hbm, out_hbm, idx_v, rows_a, rows_b, out_v,
                sem_a, sem_b, *maybe_shared):
        cid = lax.axis_index("c")
        sid = lax.axis_index("s")
        wid = sid * NC + cid
        bufs = ((rows_a, sem_a), (rows_b, sem_b))

        if shared_table is not None:
            table = maybe_shared[0]
            pltpu.sync_copy(
                table_hbm.at[pl.ds(sid * stage_rows, stage_rows)],
                table.at[pl.ds(sid * stage_rows, stage_rows)],
            )
            plsc.subcore_barrier()
        else:
            table = table_hbm

        def work():
            if whole_idx:
                pltpu.sync_copy(idx_hbm, idx_v)
                grp_off = wid * grp_per_worker
            else:
                pltpu.sync_copy(
                    idx_hbm.at[pl.ds(wid * grp_per_worker, grp_per_worker)],
                    idx_v,
                )
                grp_off = 0

            def fire(t, buf):
                rows, sem = bufs[buf]
                for g in range(G):
                    pltpu.async_copy(
                        table.at[idx_v.at[grp_off + t * G + g]],
                        rows.at[pl.ds(g * 128, 128)],
                        sem,
                    )

            def drain(buf):
                rows, sem = bufs[buf]
                for g in range(G):
                    pltpu.make_async_copy(
                        table.at[idx_v.at[grp_off + g]],
                        rows.at[pl.ds(g * 128, 128)],
                        sem,
                    ).wait()

            def compute(t, buf, off):
                rows, _ = bufs[buf]

                def node(n, carry2):
                    rbase = n * Sp
                    for k in range(D // 16):
                        sl = pl.ds(k * 16, 16)
                        lanes = min(4, S)
                        accs = [rows[rbase + j, sl] for j in range(lanes)]
                        for j in range(lanes, S):
                            accs[j % lanes] = accs[j % lanes] + rows[rbase + j, sl]
                        acc = (accs[0] + accs[1]) + (accs[2] + accs[3]) \
                            if lanes == 4 else sum(accs[1:], accs[0])
                        if chunk_writes:
                            out_v[off + n, sl] = acc
                        else:
                            out_v[t * C + n, sl] = acc
                    return carry2

                lax.fori_loop(0, C, node, 0)

            fire(0, 0)

            def pair(u, carry):
                t0 = u * 2
                fire(t0 + 1, 1)
                drain(0)
                compute(t0, 0, 0)
                pl.when(t0 + 2 < n_chunks)(lambda: fire(t0 + 2, 0))
                drain(1)
                compute(t0 + 1, 1, C)
                if chunk_writes:
                    pltpu.sync_copy(
                        out_v,
                        out_hbm.at[pl.ds(wid * R + u * pair_rows, pair_rows)],
                    )
                return carry

            lax.fori_loop(0, n_chunks // 2, pair, 0)
            if not chunk_writes:
                pltpu.sync_copy(out_v, out_hbm.at[pl.ds(wid * R, R)])

        if n_active < NW:
            pl.when(wid < n_active)(work)
        else:
            work()

    return seg_sum


def _make_sc_gather_add(n_out, S, Cg, shared_table=None, n_active=None, d=D):
    info = plsc.get_sparse_core_info()
    NC, NS = info.num_cores, info.num_subcores
    NW = NC * NS
    if n_active is None:
        n_active = NW
    assert n_out % n_active == 0
    R = n_out // n_active
    assert R % Cg == 0 and Cg % 8 == 0
    n_chunks = R // Cg
    assert n_chunks % 2 == 0
    rows_per_worker = n_chunks * S
    assert (rows_per_worker % 8) == 0

    scratch = [
        pltpu.VMEM((rows_per_worker, Cg), jnp.int32),
        pltpu.VMEM((Cg, d), jnp.float32),
        pltpu.VMEM((Cg, d), jnp.float32),
        pltpu.SemaphoreType.DMA,
        pltpu.SemaphoreType.DMA,
    ]
    if shared_table is not None:
        assert shared_table % NS == 0
        stage_rows = shared_table // NS
        scratch.append(pltpu.VMEM_SHARED((shared_table, d), jnp.float32))

    mesh = plsc.VectorSubcoreMesh(core_axis_name="c", subcore_axis_name="s")

    @functools.partial(
        pl.kernel,
        mesh=mesh,
        out_type=jax.ShapeDtypeStruct((n_out, d), jnp.float32),
        scratch_types=scratch,
    )
    def seg_sum(table_hbm, idx_hbm, out_hbm, idx_v, acc_a, acc_b,
                sem_a, sem_b, *maybe_shared):
        cid = lax.axis_index("c")
        sid = lax.axis_index("s")
        wid = sid * NC + cid
        bufs = ((acc_a, sem_a), (acc_b, sem_b))

        if shared_table is not None:
            table = maybe_shared[0]
            pltpu.sync_copy(
                table_hbm.at[pl.ds(sid * stage_rows, stage_rows)],
                table.at[pl.ds(sid * stage_rows, stage_rows)],
            )
            plsc.subcore_barrier()
        else:
            table = table_hbm

        zval = jnp.zeros((16,), jnp.float32)

        def work():
            pltpu.sync_copy(
                idx_hbm.at[pl.ds(wid * rows_per_worker, rows_per_worker)],
                idx_v,
            )

            def zero(buf):
                acc, _ = bufs[buf]

                def zrow(n, carry):
                    for k in range(d // 16):
                        acc[n, pl.ds(k * 16, 16)] = zval
                    return carry

                lax.fori_loop(0, Cg, zrow, 0)

            def fire(t, buf):
                acc, sem = bufs[buf]
                for j in range(S):
                    pltpu.async_copy(
                        table.at[idx_v.at[t * S + j]], acc, sem, add=True
                    )

            def drain(buf):
                acc, sem = bufs[buf]
                for j in range(S):
                    pltpu.make_async_copy(
                        table.at[idx_v.at[j]], acc, sem
                    ).wait()

            def flush(t, buf):
                acc, _ = bufs[buf]
                pltpu.sync_copy(acc, out_hbm.at[pl.ds(wid * R + t * Cg, Cg)])

            zero(0)
            zero(1)
            fire(0, 0)
            fire(1, 1)

            def pair(u, carry):
                t0 = u * 2
                drain(0)
                flush(t0, 0)
                zero(0)
                pl.when(t0 + 2 < n_chunks)(lambda: fire(t0 + 2, 0))
                drain(1)
                flush(t0 + 1, 1)
                zero(1)
                pl.when(t0 + 3 < n_chunks)(lambda: fire(t0 + 3, 1))
                return carry

            lax.fori_loop(0, n_chunks // 2, pair, 0)

        if n_active < NW:
            pl.when(wid < n_active)(work)
        else:
            work()

    return seg_sum


def _mm_relu_body(h_ref, s_ref, wa_ref, wb_ref, o_ref):
    o_ref[...] = jnp.maximum(
        jnp.dot(h_ref[...], wa_ref[...], preferred_element_type=jnp.float32)
        + jnp.dot(s_ref[...], wb_ref[...], preferred_element_type=jnp.float32),
        0.0,
    )


def _mm_relu(h, s, wa, wb, block_rows=512):
    n = h.shape[0]
    grid = n // block_rows
    return pl.pallas_call(
        _mm_relu_body,
        grid=(grid,),
        in_specs=[
            pl.BlockSpec((block_rows, D), lambda i: (i, 0)),
            pl.BlockSpec((block_rows, D), lambda i: (i, 0)),
            pl.BlockSpec((D, D), lambda i: (0, 0)),
            pl.BlockSpec((D, D), lambda i: (0, 0)),
        ],
        out_specs=pl.BlockSpec((block_rows, D), lambda i: (i, 0)),
        out_shape=jax.ShapeDtypeStruct((n, D), jnp.float32),
    )(h, s, wa, wb)


DC = 128


def _mm_relu_cls_body(h_ref, s_ref, wa_ref, wb_ref, wc_ref, bc_ref, y_ref):
    h = jnp.maximum(
        jnp.dot(h_ref[...], wa_ref[...], preferred_element_type=jnp.float32)
        + jnp.dot(s_ref[...], wb_ref[...], preferred_element_type=jnp.float32),
        0.0,
    )
    y_ref[...] = (
        jnp.dot(h, wc_ref[...], preferred_element_type=jnp.float32)
        + bc_ref[0:1, :]
    )


def _mm_relu_cls(h, s, wa, wb, wc, bc, block_rows=512):
    n = h.shape[0]
    grid = n // block_rows
    return pl.pallas_call(
        _mm_relu_cls_body,
        grid=(grid,),
        in_specs=[
            pl.BlockSpec((block_rows, D), lambda i: (i, 0)),
            pl.BlockSpec((block_rows, D), lambda i: (i, 0)),
            pl.BlockSpec((D, D), lambda i: (0, 0)),
            pl.BlockSpec((D, D), lambda i: (0, 0)),
            pl.BlockSpec((D, DC), lambda i: (0, 0)),
            pl.BlockSpec((8, DC), lambda i: (0, 0)),
        ],
        out_specs=pl.BlockSpec((block_rows, DC), lambda i: (i, 0)),
        out_shape=jax.ShapeDtypeStruct((n, DC), jnp.float32),
    )(h, s, wa, wb, wc, bc)


def _ls_body(p_ref, o_ref):
    logits = p_ref[...] * (1.0 / NN)
    m = jnp.max(logits, axis=1, keepdims=True)
    lse = m + jnp.log(jnp.sum(jnp.exp(logits - m), axis=1, keepdims=True))
    o_ref[...] = logits - lse


def _log_softmax(p):
    return pl.pallas_call(
        _ls_body,
        grid=(1,),
        in_specs=[pl.BlockSpec((N_GRAPHS, DC), lambda i: (0, 0))],
        out_specs=pl.BlockSpec((N_GRAPHS, DC), lambda i: (0, 0)),
        out_shape=jax.ShapeDtypeStruct((N_GRAPHS, DC), jnp.float32),
    )(p)


def kernel(target_samples, X, neigh_idx, W1, W2, W, b):
    NP = 10240

    CG = 80

    Xp = jnp.pad(X, ((0, NP - N_NODES), (0, 0)))
    idx_t = (
        jnp.pad(neigh_idx.astype(jnp.int32), ((0, NP - N_NODES), (0, 0)))
        .reshape(NP // CG, CG, M)
        .transpose(0, 2, 1)
        .reshape(-1, CG)
    )

    layer_sum = _make_sc_gather_add(n_out=NP, S=M, Cg=CG, shared_table=NP)
    pool_sum = _make_sc_gather_add(n_out=N_GRAPHS, S=NN, Cg=8, n_active=4,
                                   d=DC)

    n_cls = W.shape[1]
    wc = jnp.zeros((D, DC), jnp.float32).at[:, :n_cls].set(W)
    bc = jnp.full((8, DC), -1e4, jnp.float32).at[:, :n_cls].set(b)

    s1 = layer_sum(Xp, idx_t)
    h1 = _mm_relu(Xp, s1, W1[:D], W1[D:] * (1.0 / M))
    s2 = layer_sum(h1, idx_t)
    y = _mm_relu_cls(h1, s2, W2[:D], W2[D:] * (1.0 / M), wc, bc)

    pool_idx = (
        target_samples.astype(jnp.int32)
        .reshape(N_GRAPHS // 8, 8, NN)
        .transpose(0, 2, 1)
        .reshape(-1, 8)
    )
    p = pool_sum(y, pool_idx)
    out = _log_softmax(p)
    return out[:, :n_cls]


# speedup vs baseline: 8.3845x; 1.0704x over previous
import functools

import jax
import jax.numpy as jnp
from jax import lax
from jax.experimental import pallas as pl
from jax.experimental.pallas import tpu as pltpu
from jax.experimental.pallas import tpu_sc as plsc

N_NODES = 10000
M = 32
D = 128
NN = 100
N_GRAPHS = 64


def _make_sc_segment_sum(n_out, S, Sp, C, n_active=None, shared_table=None):
    info = plsc.get_sparse_core_info()
    NC, NS = info.num_cores, info.num_subcores
    NW = NC * NS
    if n_active is None:
        n_active = NW
    assert n_out % n_active == 0
    R = n_out // n_active
    assert R % C == 0 and R % 8 == 0
    n_chunks = R // C
    G = (C * Sp) // 128
    assert C * Sp % 128 == 0
    n_grp = (n_out * Sp) // 128
    grp_per_worker = (R * Sp) // 128
    whole_idx = (grp_per_worker % 8) != 0
    idx_rows = n_grp if whole_idx else grp_per_worker

    assert n_chunks % 2 == 0
    pair_rows = 2 * C
    assert pair_rows % 8 == 0 or n_active < NW
    chunk_writes = pair_rows % 8 == 0
    out_rows = pair_rows if chunk_writes else R
    mesh = plsc.VectorSubcoreMesh(core_axis_name="c", subcore_axis_name="s")

    scratch = [
        pltpu.VMEM((idx_rows, 128), jnp.int32),
        pltpu.VMEM((C * Sp, D), jnp.float32),
        pltpu.VMEM((C * Sp, D), jnp.float32),
        pltpu.VMEM((out_rows, D), jnp.float32),
        pltpu.SemaphoreType.DMA,
        pltpu.SemaphoreType.DMA,
    ]
    if shared_table is not None:
        assert shared_table % NS == 0
        stage_rows = shared_table // NS
        scratch.append(pltpu.VMEM_SHARED((shared_table, D), jnp.float32))

    @functools.partial(
        pl.kernel,
        mesh=mesh,
        out_type=jax.ShapeDtypeStruct((n_out, D), jnp.float32),
        scratch_types=scratch,
    )
    def seg_sum(table_hbm, idx_hbm, out_hbm, idx_v, rows_a, rows_b, out_v,
                sem_a, sem_b, *maybe_shared):
        cid = lax.axis_index("c")
        sid = lax.axis_index("s")
        wid = sid * NC + cid
        bufs = ((rows_a, sem_a), (rows_b, sem_b))

        if shared_table is not None:
            table = maybe_shared[0]
            pltpu.sync_copy(
                table_hbm.at[pl.ds(sid * stage_rows, stage_rows)],
                table.at[pl.ds(sid * stage_rows, stage_rows)],
            )
            plsc.subcore_barrier()
        else:
            table = table_hbm

        def work():
            if whole_idx:
                pltpu.sync_copy(idx_hbm, idx_v)
                grp_off = wid * grp_per_worker
            else:
                pltpu.sync_copy(
                    idx_hbm.at[pl.ds(wid * grp_per_worker, grp_per_worker)],
                    idx_v,
                )
                grp_off = 0

            def fire(t, buf):
                rows, sem = bufs[buf]
                for g in range(G):
                    pltpu.async_copy(
                        table.at[idx_v.at[grp_off + t * G + g]],
                        rows.at[pl.ds(g * 128, 128)],
                        sem,
                    )

            def drain(buf):
                rows, sem = bufs[buf]
                for g in range(G):
                    pltpu.make_async_copy(
                        table.at[idx_v.at[grp_off + g]],
                        rows.at[pl.ds(g * 128, 128)],
                        sem,
                    ).wait()

            def compute(t, buf, off):
                rows, _ = bufs[buf]

                def node(n, carry2):
                    rbase = n * Sp
                    for k in range(D // 16):
                        sl = pl.ds(k * 16, 16)
                        lanes = min(4, S)
                        accs = [rows[rbase + j, sl] for j in range(lanes)]
                        for j in range(lanes, S):
                            accs[j % lanes] = accs[j % lanes] + rows[rbase + j, sl]
                        acc = (accs[0] + accs[1]) + (accs[2] + accs[3]) \
                            if lanes == 4 else sum(accs[1:], accs[0])
                        if chunk_writes:
                            out_v[off + n, sl] = acc
                        else:
                            out_v[t * C + n, sl] = acc
                    return carry2

                lax.fori_loop(0, C, node, 0)

            fire(0, 0)

            def pair(u, carry):
                t0 = u * 2
                fire(t0 + 1, 1)
                drain(0)
                compute(t0, 0, 0)
                pl.when(t0 + 2 < n_chunks)(lambda: fire(t0 + 2, 0))
                drain(1)
                compute(t0 + 1, 1, C)
                if chunk_writes:
                    pltpu.sync_copy(
                        out_v,
                        out_hbm.at[pl.ds(wid * R + u * pair_rows, pair_rows)],
                    )
                return carry

            lax.fori_loop(0, n_chunks // 2, pair, 0)
            if not chunk_writes:
                pltpu.sync_copy(out_v, out_hbm.at[pl.ds(wid * R, R)])

        if n_active < NW:
            pl.when(wid < n_active)(work)
        else:
            work()

    return seg_sum


def _make_sc_gather_add(n_out, S, Cg, shared_table=None, n_active=None, d=D):
    info = plsc.get_sparse_core_info()
    NC, NS = info.num_cores, info.num_subcores
    NW = NC * NS
    if n_active is None:
        n_active = NW
    assert n_out % n_active == 0
    R = n_out // n_active
    assert R % Cg == 0 and Cg % 8 == 0
    n_chunks = R // Cg
    assert n_chunks == 1 or n_chunks % 2 == 0
    rows_per_worker = n_chunks * S
    whole_idx = (rows_per_worker % 8) != 0 or ((rows_per_worker * Cg) % 8) != 0
    idx_rows = n_active * rows_per_worker if whole_idx else rows_per_worker

    scratch = [
        pltpu.VMEM((idx_rows, Cg), jnp.int32),
        pltpu.VMEM((Cg, d), jnp.float32),
        pltpu.VMEM((Cg, d), jnp.float32),
        pltpu.SemaphoreType.DMA,
        pltpu.SemaphoreType.DMA,
    ]
    if shared_table is not None:
        assert shared_table % NS == 0
        stage_rows = shared_table // NS
        scratch.append(pltpu.VMEM_SHARED((shared_table, d), jnp.float32))

    mesh = plsc.VectorSubcoreMesh(core_axis_name="c", subcore_axis_name="s")

    @functools.partial(
        pl.kernel,
        mesh=mesh,
        out_type=jax.ShapeDtypeStruct((n_out, d), jnp.float32),
        scratch_types=scratch,
    )
    def seg_sum(table_hbm, idx_hbm, out_hbm, idx_v, acc_a, acc_b,
                sem_a, sem_b, *maybe_shared):
        cid = lax.axis_index("c")
        sid = lax.axis_index("s")
        wid = sid * NC + cid
        bufs = ((acc_a, sem_a), (acc_b, sem_b))

        if shared_table is not None:
            table = maybe_shared[0]
            pltpu.sync_copy(
                table_hbm.at[pl.ds(sid * stage_rows, stage_rows)],
                table.at[pl.ds(sid * stage_rows, stage_rows)],
            )
            plsc.subcore_barrier()
        else:
            table = table_hbm

        zval = jnp.zeros((16,), jnp.float32)

        def work():
            if whole_idx:
                pltpu.sync_copy(idx_hbm, idx_v)
                row_off = wid * rows_per_worker
            else:
                pltpu.sync_copy(
                    idx_hbm.at[pl.ds(wid * rows_per_worker, rows_per_worker)],
                    idx_v,
                )
                row_off = 0

            def zero(buf):
                acc, _ = bufs[buf]

                def zrow(n, carry):
                    for k in range(d // 16):
                        acc[n, pl.ds(k * 16, 16)] = zval
                    return carry

                lax.fori_loop(0, Cg, zrow, 0)

            def fire(t, buf):
                acc, sem = bufs[buf]
                for j in range(S):
                    pltpu.async_copy(
                        table.at[idx_v.at[row_off + t * S + j]], acc, sem,
                        add=True,
                    )

            def drain(buf):
                acc, sem = bufs[buf]
                for j in range(S):
                    pltpu.make_async_copy(
                        table.at[idx_v.at[row_off + j]], acc, sem
                    ).wait()

            def flush(t, buf):
                acc, _ = bufs[buf]
                pltpu.sync_copy(acc, out_hbm.at[pl.ds(wid * R + t * Cg, Cg)])

            if n_chunks == 1:
                zero(0)
                fire(0, 0)
                drain(0)
                flush(0, 0)
                return

            zero(0)
            zero(1)
            fire(0, 0)
            fire(1, 1)

            def pair(u, carry):
                t0 = u * 2
                drain(0)
                flush(t0, 0)
                zero(0)
                pl.when(t0 + 2 < n_chunks)(lambda: fire(t0 + 2, 0))
                drain(1)
                flush(t0 + 1, 1)
                zero(1)
                pl.when(t0 + 3 < n_chunks)(lambda: fire(t0 + 3, 1))
                return carry

            lax.fori_loop(0, n_chunks // 2, pair, 0)

        if n_active < NW:
            pl.when(wid < n_active)(work)
        else:
            work()

    return seg_sum


def _mm_relu_body(h_ref, s_ref, wa_ref, wb_ref, o_ref):
    o_ref[...] = jnp.maximum(
        jnp.dot(h_ref[...], wa_ref[...], preferred_element_type=jnp.float32)
        + jnp.dot(s_ref[...], wb_ref[...], preferred_element_type=jnp.float32),
        0.0,
    )


def _mm_relu(h, s, wa, wb, block_rows=1024):
    n = h.shape[0]
    grid = n // block_rows
    return pl.pallas_call(
        _mm_relu_body,
        grid=(grid,),
        in_specs=[
            pl.BlockSpec((block_rows, D), lambda i: (i, 0)),
            pl.BlockSpec((block_rows, D), lambda i: (i, 0)),
            pl.BlockSpec((D, D), lambda i: (0, 0)),
            pl.BlockSpec((D, D), lambda i: (0, 0)),
        ],
        out_specs=pl.BlockSpec((block_rows, D), lambda i: (i, 0)),
        out_shape=jax.ShapeDtypeStruct((n, D), jnp.float32),
    )(h, s, wa, wb)


DC = 128


def _mm_relu_cls_body(h_ref, s_ref, wa_ref, wb_ref, wc_ref, bc_ref, y_ref):
    h = jnp.maximum(
        jnp.dot(h_ref[...], wa_ref[...], preferred_element_type=jnp.float32)
        + jnp.dot(s_ref[...], wb_ref[...], preferred_element_type=jnp.float32),
        0.0,
    )
    y_ref[...] = (
        jnp.dot(h, wc_ref[...], preferred_element_type=jnp.float32)
        + bc_ref[0:1, :]
    )


def _mm_relu_cls(h, s, wa, wb, wc, bc, block_rows=1024):
    n = h.shape[0]
    grid = n // block_rows
    return pl.pallas_call(
        _mm_relu_cls_body,
        grid=(grid,),
        in_specs=[
            pl.BlockSpec((block_rows, D), lambda i: (i, 0)),
            pl.BlockSpec((block_rows, D), lambda i: (i, 0)),
            pl.BlockSpec((D, D), lambda i: (0, 0)),
            pl.BlockSpec((D, D), lambda i: (0, 0)),
            pl.BlockSpec((D, DC), lambda i: (0, 0)),
            pl.BlockSpec((8, DC), lambda i: (0, 0)),
        ],
        out_specs=pl.BlockSpec((block_rows, DC), lambda i: (i, 0)),
        out_shape=jax.ShapeDtypeStruct((n, DC), jnp.float32),
    )(h, s, wa, wb, wc, bc)


def _ls_body(p_ref, o_ref):
    logits = p_ref[...] * (1.0 / NN)
    m = jnp.max(logits, axis=1, keepdims=True)
    lse = m + jnp.log(jnp.sum(jnp.exp(logits - m), axis=1, keepdims=True))
    o_ref[...] = logits - lse


def _log_softmax(p):
    return pl.pallas_call(
        _ls_body,
        grid=(1,),
        in_specs=[pl.BlockSpec((N_GRAPHS, DC), lambda i: (0, 0))],
        out_specs=pl.BlockSpec((N_GRAPHS, DC), lambda i: (0, 0)),
        out_shape=jax.ShapeDtypeStruct((N_GRAPHS, DC), jnp.float32),
    )(p)


def kernel(target_samples, X, neigh_idx, W1, W2, W, b):
    NP = 10240

    CG = 80

    Xp = jnp.pad(X, ((0, NP - N_NODES), (0, 0)))
    idx_t = (
        jnp.pad(neigh_idx.astype(jnp.int32), ((0, NP - N_NODES), (0, 0)))
        .reshape(NP // CG, CG, M)
        .transpose(0, 2, 1)
        .reshape(-1, CG)
    )

    layer_sum = _make_sc_gather_add(n_out=NP, S=M, Cg=CG, shared_table=NP)
    pool_sum = _make_sc_gather_add(n_out=N_GRAPHS, S=NN, Cg=8, n_active=8,
                                   d=DC)

    n_cls = W.shape[1]
    wc = jnp.zeros((D, DC), jnp.float32).at[:, :n_cls].set(W)
    bc = jnp.full((8, DC), -1e4, jnp.float32).at[:, :n_cls].set(b)

    s1 = layer_sum(Xp, idx_t)
    h1 = _mm_relu(Xp, s1, W1[:D], W1[D:] * (1.0 / M))
    s2 = layer_sum(h1, idx_t)
    y = _mm_relu_cls(h1, s2, W2[:D], W2[D:] * (1.0 / M), wc, bc)

    pool_idx = (
        target_samples.astype(jnp.int32)
        .reshape(N_GRAPHS // 8, 8, NN)
        .transpose(0, 2, 1)
        .reshape(-1, 8)
    )
    p = pool_sum(y, pool_idx)
    out = _log_softmax(p)
    return out[:, :n_cls]


# speedup vs baseline: 8.3991x; 1.0017x over previous
import functools

import jax
import jax.numpy as jnp
from jax import lax
from jax.experimental import pallas as pl
from jax.experimental.pallas import tpu as pltpu
from jax.experimental.pallas import tpu_sc as plsc

N_NODES = 10000
M = 32
D = 128
NN = 100
N_GRAPHS = 64


def _make_sc_gather_add(n_out, S, Cg, shared_table=None, n_active=None, d=D):
    info = plsc.get_sparse_core_info()
    NC, NS = info.num_cores, info.num_subcores
    NW = NC * NS
    if n_active is None:
        n_active = NW
    assert n_out % n_active == 0
    R = n_out // n_active
    assert R % Cg == 0 and Cg % 8 == 0
    n_chunks = R // Cg
    assert n_chunks == 1 or n_chunks % 2 == 0
    rows_per_worker = n_chunks * S
    whole_idx = (rows_per_worker % 8) != 0 or ((rows_per_worker * Cg) % 8) != 0
    idx_rows = n_active * rows_per_worker if whole_idx else rows_per_worker

    scratch = [
        pltpu.VMEM((idx_rows, Cg), jnp.int32),
        pltpu.VMEM((Cg, d), jnp.float32),
        pltpu.VMEM((Cg, d), jnp.float32),
        pltpu.SemaphoreType.DMA,
        pltpu.SemaphoreType.DMA,
    ]
    if shared_table is not None:
        assert shared_table % NS == 0
        stage_rows = shared_table // NS
        scratch.append(pltpu.VMEM_SHARED((shared_table, d), jnp.float32))

    mesh = plsc.VectorSubcoreMesh(core_axis_name="c", subcore_axis_name="s")

    @functools.partial(
        pl.kernel,
        mesh=mesh,
        out_type=jax.ShapeDtypeStruct((n_out, d), jnp.float32),
        scratch_types=scratch,
    )
    def seg_sum(table_hbm, idx_hbm, out_hbm, idx_v, acc_a, acc_b,
                sem_a, sem_b, *maybe_shared):
        cid = lax.axis_index("c")
        sid = lax.axis_index("s")
        wid = sid * NC + cid
        bufs = ((acc_a, sem_a), (acc_b, sem_b))

        if shared_table is not None:
            table = maybe_shared[0]
            pltpu.sync_copy(
                table_hbm.at[pl.ds(sid * stage_rows, stage_rows)],
                table.at[pl.ds(sid * stage_rows, stage_rows)],
            )
            plsc.subcore_barrier()
        else:
            table = table_hbm

        zval = jnp.zeros((16,), jnp.float32)

        def work():
            if whole_idx:
                pltpu.sync_copy(idx_hbm, idx_v)
                row_off = wid * rows_per_worker
            else:
                pltpu.sync_copy(
                    idx_hbm.at[pl.ds(wid * rows_per_worker, rows_per_worker)],
                    idx_v,
                )
                row_off = 0

            def zero(buf):
                acc, _ = bufs[buf]

                def zrow(n, carry):
                    for k in range(d // 16):
                        acc[n, pl.ds(k * 16, 16)] = zval
                    return carry

                lax.fori_loop(0, Cg, zrow, 0)

            def fire(t, buf):
                acc, sem = bufs[buf]
                for j in range(S):
                    pltpu.async_copy(
                        table.at[idx_v.at[row_off + t * S + j]], acc, sem,
                        add=True,
                    )

            def drain(buf):
                acc, sem = bufs[buf]
                for j in range(S):
                    pltpu.make_async_copy(
                        table.at[idx_v.at[row_off + j]], acc, sem
                    ).wait()

            def flush(t, buf):
                acc, _ = bufs[buf]
                pltpu.sync_copy(acc, out_hbm.at[pl.ds(wid * R + t * Cg, Cg)])

            if n_chunks == 1:
                zero(0)
                fire(0, 0)
                drain(0)
                flush(0, 0)
                return

            zero(0)
            zero(1)
            fire(0, 0)
            fire(1, 1)

            def pair(u, carry):
                t0 = u * 2
                drain(0)
                flush(t0, 0)
                zero(0)
                pl.when(t0 + 2 < n_chunks)(lambda: fire(t0 + 2, 0))
                drain(1)
                flush(t0 + 1, 1)
                zero(1)
                pl.when(t0 + 3 < n_chunks)(lambda: fire(t0 + 3, 1))
                return carry

            lax.fori_loop(0, n_chunks // 2, pair, 0)

        if n_active < NW:
            pl.when(wid < n_active)(work)
        else:
            work()

    return seg_sum


def _mm_relu_body(h_ref, s_ref, wa_ref, wb_ref, o_ref):
    o_ref[...] = jnp.maximum(
        jnp.dot(h_ref[...], wa_ref[...], preferred_element_type=jnp.float32)
        + jnp.dot(s_ref[...], wb_ref[...], preferred_element_type=jnp.float32),
        0.0,
    )


def _mm_relu(h, s, wa, wb, block_rows=1024):
    n = h.shape[0]
    grid = n // block_rows
    return pl.pallas_call(
        _mm_relu_body,
        grid=(grid,),
        in_specs=[
            pl.BlockSpec((block_rows, D), lambda i: (i, 0)),
            pl.BlockSpec((block_rows, D), lambda i: (i, 0)),
            pl.BlockSpec((D, D), lambda i: (0, 0)),
            pl.BlockSpec((D, D), lambda i: (0, 0)),
        ],
        out_specs=pl.BlockSpec((block_rows, D), lambda i: (i, 0)),
        out_shape=jax.ShapeDtypeStruct((n, D), jnp.float32),
    )(h, s, wa, wb)


DC = 128


def _mm_relu_cls_body(h_ref, s_ref, wa_ref, wb_ref, wc_ref, bc_ref, y_ref):
    h = jnp.maximum(
        jnp.dot(h_ref[...], wa_ref[...], preferred_element_type=jnp.float32)
        + jnp.dot(s_ref[...], wb_ref[...], preferred_element_type=jnp.float32),
        0.0,
    )
    y_ref[...] = (
        jnp.dot(h, wc_ref[...], preferred_element_type=jnp.float32)
        + bc_ref[0:1, :]
    )


def _mm_relu_cls(h, s, wa, wb, wc, bc, block_rows=1024):
    n = h.shape[0]
    grid = n // block_rows
    return pl.pallas_call(
        _mm_relu_cls_body,
        grid=(grid,),
        in_specs=[
            pl.BlockSpec((block_rows, D), lambda i: (i, 0)),
            pl.BlockSpec((block_rows, D), lambda i: (i, 0)),
            pl.BlockSpec((D, D), lambda i: (0, 0)),
            pl.BlockSpec((D, D), lambda i: (0, 0)),
            pl.BlockSpec((D, DC), lambda i: (0, 0)),
            pl.BlockSpec((8, DC), lambda i: (0, 0)),
        ],
        out_specs=pl.BlockSpec((block_rows, DC), lambda i: (i, 0)),
        out_shape=jax.ShapeDtypeStruct((n, DC), jnp.float32),
    )(h, s, wa, wb, wc, bc)


def _ls_body(p_ref, o_ref):
    logits = p_ref[...] * (1.0 / NN)
    m = jnp.max(logits, axis=1, keepdims=True)
    lse = m + jnp.log(jnp.sum(jnp.exp(logits - m), axis=1, keepdims=True))
    o_ref[...] = logits - lse


def _log_softmax(p):
    return pl.pallas_call(
        _ls_body,
        grid=(1,),
        in_specs=[pl.BlockSpec((N_GRAPHS, DC), lambda i: (0, 0))],
        out_specs=pl.BlockSpec((N_GRAPHS, DC), lambda i: (0, 0)),
        out_shape=jax.ShapeDtypeStruct((N_GRAPHS, DC), jnp.float32),
    )(p)


def kernel(target_samples, X, neigh_idx, W1, W2, W, b):
    NP = 10240

    CG = 80

    Xp = jnp.pad(X, ((0, NP - N_NODES), (0, 0)))
    idx_t = (
        jnp.pad(neigh_idx.astype(jnp.int32), ((0, NP - N_NODES), (0, 0)))
        .reshape(NP // CG, CG, M)
        .transpose(0, 2, 1)
        .reshape(-1, CG)
    )

    layer_sum = _make_sc_gather_add(n_out=NP, S=M, Cg=CG, shared_table=NP)
    pool_sum = _make_sc_gather_add(n_out=N_GRAPHS, S=NN, Cg=8, n_active=8,
                                   d=DC)

    n_cls = W.shape[1]
    wc = jnp.zeros((D, DC), jnp.float32).at[:, :n_cls].set(W)
    bc = jnp.full((8, DC), -1e4, jnp.float32).at[:, :n_cls].set(b)

    s1 = layer_sum(Xp, idx_t)
    h1 = _mm_relu(Xp, s1, W1[:D], W1[D:] * (1.0 / M))
    s2 = layer_sum(h1, idx_t)
    y = _mm_relu_cls(h1, s2, W2[:D], W2[D:] * (1.0 / M), wc, bc)

    pool_idx = (
        target_samples.astype(jnp.int32)
        .reshape(N_GRAPHS // 8, 8, NN)
        .transpose(0, 2, 1)
        .reshape(-1, 8)
    )
    p = pool_sum(y, pool_idx)
    out = _log_softmax(p)
    return out[:, :n_cls]
